# baseline (speedup 1.0000x reference)
import functools

import jax
import jax.numpy as jnp
from jax import lax
from jax.experimental import pallas as pl
from jax.experimental.pallas import tpu as pltpu
from jax.experimental.pallas import tpu_sc as plsc

NC, NS = 2, 16
NW = NC * NS
CHUNK = 128
NBUF = 4


@functools.lru_cache(maxsize=None)
def _build(n_chunk: int, D: int):
    mesh = plsc.VectorSubcoreMesh(
        core_axis_name="c", subcore_axis_name="s",
        num_cores=NC, num_subcores=NS)

    @functools.partial(
        pl.kernel,
        out_type=jax.ShapeDtypeStruct((NW, n_chunk, CHUNK, D), jnp.float32),
        mesh=mesh,
        scratch_types=[
            pltpu.VMEM((n_chunk, CHUNK), jnp.int32),
            [pltpu.VMEM((CHUNK, D), jnp.float32) for _ in range(NBUF)],
            [pltpu.SemaphoreType.DMA for _ in range(NBUF)],
            [pltpu.SemaphoreType.DMA for _ in range(NBUF)],
        ],
        compiler_params=pltpu.CompilerParams(use_tc_tiling_on_sc=False),
    )
    def k(idx_hbm, table_hbm, out_hbm, idx_v, rows, gsem, wsem):
        wid = lax.axis_index("s") * NC + lax.axis_index("c")

        pltpu.sync_copy(idx_hbm.at[wid], idx_v)

        def gather(g, b):
            pltpu.async_copy(table_hbm.at[idx_v.at[g]], rows[b], gsem[b])

        def wait_gather(b):
            pltpu.make_async_copy(
                table_hbm.at[idx_v.at[0]], rows[b], gsem[b]).wait()

        def writeback(g, b):
            pltpu.async_copy(rows[b], out_hbm.at[wid, g], wsem[b])

        def wait_writeback(b):
            pltpu.make_async_copy(rows[b], out_hbm.at[wid, 0], wsem[b]).wait()

        for b in range(NBUF):
            gather(b, b)

        def body(outer, carry):
            for b in range(NBUF):
                g = outer * NBUF + b
                wait_gather(b)
                writeback(g, b)
                wait_writeback(b)
                gather(g + NBUF, b)
            return carry

        lax.fori_loop(0, (n_chunk - NBUF) // NBUF, body, 0)

        for b in range(NBUF):
            g = n_chunk - NBUF + b
            wait_gather(b)
            writeback(g, b)
        for b in range(NBUF):
            wait_writeback(b)

    return k


def kernel(x, table):
    B, H = x.shape
    V, D = table.shape
    total = B * H
    n_chunk = total // (NW * CHUNK)
    idx = x.reshape(NW, n_chunk, CHUNK).astype(jnp.int32)
    out = _build(n_chunk, D)(idx, table)
    return out.reshape(B, H, D)


# speedup vs baseline: 1.0032x; 1.0032x over previous
import functools

import jax
import jax.numpy as jnp
from jax import lax
from jax.experimental import pallas as pl
from jax.experimental.pallas import tpu as pltpu
from jax.experimental.pallas import tpu_sc as plsc

NC, NS = 2, 16
NW = NC * NS
CHUNK = 128
NBUF = 8
AHEAD = 4


@functools.lru_cache(maxsize=None)
def _build(n_chunk: int, D: int):
    mesh = plsc.VectorSubcoreMesh(
        core_axis_name="c", subcore_axis_name="s",
        num_cores=NC, num_subcores=NS)

    @functools.partial(
        pl.kernel,
        out_type=jax.ShapeDtypeStruct((NW, n_chunk, CHUNK, D), jnp.float32),
        mesh=mesh,
        scratch_types=[
            pltpu.VMEM((n_chunk, CHUNK), jnp.int32),
            [pltpu.VMEM((CHUNK, D), jnp.float32) for _ in range(NBUF)],
            [pltpu.SemaphoreType.DMA for _ in range(NBUF)],
            [pltpu.SemaphoreType.DMA for _ in range(NBUF)],
        ],
        compiler_params=pltpu.CompilerParams(use_tc_tiling_on_sc=False),
    )
    def k(idx_hbm, table_hbm, out_hbm, idx_v, rows, gsem, wsem):
        wid = lax.axis_index("s") * NC + lax.axis_index("c")

        pltpu.sync_copy(idx_hbm.at[wid], idx_v)

        def gather(g, b):
            pltpu.async_copy(table_hbm.at[idx_v.at[g]], rows[b], gsem[b])

        def wait_gather(b):
            pltpu.make_async_copy(
                table_hbm.at[idx_v.at[0]], rows[b], gsem[b]).wait()

        def writeback(g, b):
            pltpu.async_copy(rows[b], out_hbm.at[wid, g], wsem[b])

        def wait_writeback(b):
            pltpu.make_async_copy(rows[b], out_hbm.at[wid, 0], wsem[b]).wait()


        for g in range(AHEAD):
            gather(g, g % NBUF)

        for g in range(NBUF - AHEAD):
            wait_gather(g % NBUF)
            writeback(g, g % NBUF)
            gather(g + AHEAD, (g + AHEAD) % NBUF)

        steady0 = NBUF - AHEAD
        n_steady = (n_chunk - AHEAD) - steady0
        assert n_steady % NBUF == 0

        def body(outer, carry):
            for j in range(NBUF):
                g = steady0 + outer * NBUF + j
                b = (steady0 + j) % NBUF
                wait_gather(b)
                writeback(g, b)
                b2 = (steady0 + j + AHEAD) % NBUF
                wait_writeback(b2)
                gather(g + AHEAD, b2)
            return carry

        lax.fori_loop(0, n_steady // NBUF, body, 0)

        for g in range(n_chunk - AHEAD, n_chunk):
            wait_gather(g % NBUF)
            writeback(g, g % NBUF)
        for b in range(NBUF):
            wait_writeback(b)

    return k


def kernel(x, table):
    B, H = x.shape
    V, D = table.shape
    total = B * H
    n_chunk = total // (NW * CHUNK)
    idx = x.reshape(NW, n_chunk, CHUNK).astype(jnp.int32)
    out = _build(n_chunk, D)(idx, table)
    return out.reshape(B, H, D)
